# Initial kernel scaffold; baseline (speedup 1.0000x reference)
import jax
import jax.numpy as jnp
from jax.experimental import pallas as pl


def kernel(xi, memory, W_rk, b_rk, W_rs, b_rs):
    raise NotImplementedError("write your pallas kernel here")


# speedup vs baseline: 4.8208x; 4.8208x over previous
import jax
import jax.numpy as jnp
from jax.experimental import pallas as pl
from jax.experimental.pallas import tpu as pltpu

_B, _M, _W, _R, _K, _IN = 64, 16384, 64, 8, 8, 1024


def _proj_body(xi_ref, wrk_ref, brk_ref, wrs_ref, brs_ref, keys_ref, str_ref):
    f32 = jnp.float32
    hi = jax.lax.Precision.HIGHEST
    xi = xi_ref[...]
    keys_ref[...] = jnp.tanh(
        jax.lax.dot_general(xi, wrk_ref[...], (((1,), (1,)), ((), ())),
                            preferred_element_type=f32)
        + brk_ref[...]
    )
    x = (jax.lax.dot_general(xi, wrs_ref[...], (((1,), (1,)), ((), ())),
                             preferred_element_type=f32)
         + brs_ref[...])
    str_ref[...] = jnp.maximum(x, 0.0) + jnp.log(1.0 + jnp.exp(-jnp.abs(x)))


def _read_body(keys_ref, str_ref, mem_ref, out_ref):
    f32 = jnp.float32
    hi = jax.lax.Precision.HIGHEST

    keys = keys_ref[0]
    strength = str_ref[0]
    mem = mem_ref[0]

    m2 = jnp.sum(mem * mem, axis=1)
    k2 = jnp.sum(keys * keys, axis=1)
    km = jax.lax.dot_general(keys, mem, (((1,), (1,)), ((), ())),
                             preferred_element_type=f32)
    dist = k2[:, None] + m2[None, :] - 2.0 * km

    iota = jax.lax.broadcasted_iota(jnp.int32, (_R, _M), 1)
    cur = dist
    d_cols, idx_cols = [], []
    for _ in range(_K):
        mv = jnp.min(cur, axis=1, keepdims=True)
        idx = jnp.min(jnp.where(cur == mv, iota, _M),
                      axis=1, keepdims=True)
        d_cols.append(mv)
        idx_cols.append(idx)
        cur = jnp.where(iota == idx, jnp.float32(jnp.inf), cur)

    d = jnp.concatenate(d_cols, axis=1)
    maxd = d[:, _K - 1:_K] + 1e-6
    logits = -(d / maxd) * strength
    logits = logits - jnp.max(logits, axis=1, keepdims=True)
    e = jnp.exp(logits)
    attn = e / jnp.sum(e, axis=1, keepdims=True)

    wmat = jnp.zeros((_R, _M), f32)
    for k in range(_K):
        wmat = wmat + jnp.where(iota == idx_cols[k], attn[:, k:k + 1], 0.0)
    out_ref[0] = jax.lax.dot_general(wmat, mem, (((1,), (0,)), ((), ())),
                                     precision=hi, preferred_element_type=f32)


@jax.jit
def kernel(xi, memory, W_rk, b_rk, W_rs, b_rs):
    keys_flat, strengths = pl.pallas_call(
        _proj_body,
        out_shape=[
            jax.ShapeDtypeStruct((_B, _R * _W), jnp.float32),
            jax.ShapeDtypeStruct((_B, _R), jnp.float32),
        ],
    )(xi, W_rk, b_rk.reshape(1, _R * _W), W_rs, b_rs.reshape(1, _R))

    keys = keys_flat.reshape(_B, _R, _W)
    strengths = strengths.reshape(_B, _R, 1)

    out = pl.pallas_call(
        _read_body,
        grid=(_B,),
        in_specs=[
            pl.BlockSpec((1, _R, _W), lambda b: (b, 0, 0)),
            pl.BlockSpec((1, _R, 1), lambda b: (b, 0, 0)),
            pl.BlockSpec((1, _M, _W), lambda b: (b, 0, 0)),
        ],
        out_specs=pl.BlockSpec((1, _R, _W), lambda b: (b, 0, 0)),
        out_shape=jax.ShapeDtypeStruct((_B, _R, _W), jnp.float32),
        compiler_params=pltpu.CompilerParams(
            dimension_semantics=("arbitrary",),
        ),
    )(keys, strengths, memory)
    return out
